# baseline (speedup 1.0000x reference)
import functools

import jax
import jax.numpy as jnp
from jax import lax
from jax.experimental import pallas as pl
from jax.experimental.pallas import tpu as pltpu
from jax.experimental.pallas import tpu_sc as plsc

VOCAB = 100000
EMBED_DIM = 64
BATCH = 4096
HIST = 50

NUM_CORES = 2
NUM_SUBCORES = 16
NUM_WORKERS = NUM_CORES * NUM_SUBCORES

TOTAL = BATCH * HIST
PER_WORKER = TOTAL // NUM_WORKERS
CHUNK = 800
NUM_CHUNKS = PER_WORKER // CHUNK
NBUF = 2
NSPLIT = 4
SUB = CHUNK // NSPLIT


_mesh = plsc.VectorSubcoreMesh(core_axis_name="c", subcore_axis_name="s")


@functools.partial(
    pl.kernel,
    out_type=jax.ShapeDtypeStruct((TOTAL, EMBED_DIM), jnp.float32),
    mesh=_mesh,
    scratch_types=(
        [pltpu.VMEM((CHUNK,), jnp.int32) for _ in range(NBUF)]
        + [pltpu.VMEM((CHUNK, EMBED_DIM), jnp.float32) for _ in range(NBUF)]
        + [pltpu.SemaphoreType.DMA for _ in range(2 * NBUF)]
        + [pltpu.SemaphoreType.DMA for _ in range(NSPLIT * NBUF)]
    ),
    compiler_params=pltpu.CompilerParams(use_tc_tiling_on_sc=False),
)
def _gather_kernel(ids_hbm, table_hbm, out_hbm,
                   idx0, idx1, rows0, rows1,
                   si0, si1, ss0, ss1,
                   *sg):
    wid = lax.axis_index("s") * NUM_CORES + lax.axis_index("c")
    base = wid * PER_WORKER

    idx = (idx0, idx1)
    rows = (rows0, rows1)
    si = (si0, si1)
    ss = (ss0, ss1)
    sg = (sg[:NSPLIT], sg[NSPLIT:])

    def off(c):
        return base + c * CHUNK

    def issue_gathers(c):
        b = c % NBUF
        return [
            pltpu.async_copy(
                table_hbm.at[idx[b].at[pl.ds(q * SUB, SUB)]],
                rows[b].at[pl.ds(q * SUB, SUB)],
                sg[b][q])
            for q in range(NSPLIT)
        ]

    idx_d, g_d, s_d = {}, {}, {}
    for c in range(min(NBUF, NUM_CHUNKS)):
        b = c % NBUF
        idx_d[c] = pltpu.async_copy(
            ids_hbm.at[pl.ds(off(c), CHUNK)], idx[b], si[b])
    idx_d[0].wait()
    g_d[0] = issue_gathers(0)

    for c in range(NUM_CHUNKS):
        b = c % NBUF
        if c + 1 < NUM_CHUNKS:
            idx_d[c + 1].wait()
            if c - 1 >= 0:
                s_d[c - 1].wait()
            g_d[c + 1] = issue_gathers(c + 1)
        for d in g_d[c]:
            d.wait()
        s_d[c] = pltpu.async_copy(
            rows[b], out_hbm.at[pl.ds(off(c), CHUNK)], ss[b])
        if c + 2 < NUM_CHUNKS:
            idx_d[c + 2] = pltpu.async_copy(
                ids_hbm.at[pl.ds(off(c + 2), CHUNK)], idx[b], si[b])

    if NUM_CHUNKS >= 2:
        s_d[NUM_CHUNKS - 2].wait()
    s_d[NUM_CHUNKS - 1].wait()


def kernel(start_ids, table):
    ids = start_ids.reshape(-1).astype(jnp.int32)
    out = _gather_kernel(ids, table)
    return out.reshape(BATCH, HIST, EMBED_DIM)
